# Initial kernel scaffold; baseline (speedup 1.0000x reference)
import jax
import jax.numpy as jnp
from jax.experimental import pallas as pl


def kernel(x, edge_index, W, b, Wd, bd):
    raise NotImplementedError("write your pallas kernel here")


# speedup vs baseline: 123.6253x; 123.6253x over previous
import functools

import jax
import jax.numpy as jnp
from jax import lax
from jax.experimental import pallas as pl
from jax.experimental.pallas import tpu as pltpu
from jax.experimental.pallas import tpu_sc as plsc

N_NODES = 10000
N_EDGES = 320000
L = 16
NC, NS = 2, 16
NW = NC * NS
E_PER_W = N_EDGES // NW

def _sc_degree_body(dst_hbm, out_hbm, dst_v, deg_v):
    wid = lax.axis_index("s") * NC + lax.axis_index("c")
    base = wid * E_PER_W
    pltpu.sync_copy(dst_hbm.at[pl.ds(base, E_PER_W)], dst_v)

    zeros = jnp.zeros((L,), jnp.float32)

    def zero_body(i, carry):
        deg_v[pl.ds(i * L, L)] = zeros
        return carry

    lax.fori_loop(0, N_NODES // L, zero_body, 0)

    ones = jnp.ones((L,), jnp.float32)

    def body(i, carry):
        idx = dst_v[pl.ds(i * L, L)]
        plsc.addupdate_scatter(deg_v, [idx], ones)
        return carry

    lax.fori_loop(0, E_PER_W // L, body, 0)
    pltpu.sync_copy(deg_v, out_hbm.at[wid])


def _sc_tsum_body(src_hbm, dst_hbm, dinv_hbm, out_hbm, src_v, dst_v, dinv_v, t_v):
    wid = lax.axis_index("s") * NC + lax.axis_index("c")
    base = wid * E_PER_W
    pltpu.sync_copy(dinv_hbm, dinv_v)
    pltpu.sync_copy(src_hbm.at[pl.ds(base, E_PER_W)], src_v)
    pltpu.sync_copy(dst_hbm.at[pl.ds(base, E_PER_W)], dst_v)

    zeros = jnp.zeros((L,), jnp.float32)

    def zero_body(i, carry):
        t_v[pl.ds(i * L, L)] = zeros
        return carry

    lax.fori_loop(0, N_NODES // L, zero_body, 0)

    def body(i, carry):
        si = src_v[pl.ds(i * L, L)]
        di = dst_v[pl.ds(i * L, L)]
        dvals = plsc.load_gather(dinv_v, [di])
        plsc.addupdate_scatter(t_v, [si], dvals)
        return carry

    lax.fori_loop(0, E_PER_W // L, body, 0)
    pltpu.sync_copy(t_v, out_hbm.at[wid])


def _tc_dinv(deg_part_ref, out_ref):
    deg = jnp.sum(deg_part_ref[...], axis=0, keepdims=True) + 1.0
    out_ref[...] = lax.rsqrt(deg)


def _tc_final(t_part_ref, dinv_ref, x_ref, w_ref, b_ref, wd_ref, bd_ref, out_ref):
    t = jnp.sum(t_part_ref[...], axis=0, keepdims=True)
    dinv = dinv_ref[...]
    c = dinv * (dinv + t)
    cx = lax.dot_general(c, x_ref[...], (((1,), (0,)), ((), ())),
                         preferred_element_type=jnp.float32)
    pooled = lax.dot_general(cx, w_ref[...], (((1,), (0,)), ((), ())),
                             preferred_element_type=jnp.float32)
    pooled = pooled + float(N_NODES) * b_ref[...]
    logits = lax.dot_general(pooled, wd_ref[...], (((1,), (0,)), ((), ())),
                             preferred_element_type=jnp.float32)
    logits = logits + bd_ref[...]
    m = jnp.max(logits, axis=1, keepdims=True)
    e = jnp.exp(logits - m)
    out_ref[...] = e / jnp.sum(e, axis=1, keepdims=True)


@functools.cache
def _build_sc_kernels():
    mesh = plsc.VectorSubcoreMesh(core_axis_name="c", subcore_axis_name="s")
    params = pltpu.CompilerParams(needs_layout_passes=False)
    sc_degree = pl.kernel(
        _sc_degree_body,
        mesh=mesh,
        out_type=jax.ShapeDtypeStruct((NW, N_NODES), jnp.float32),
        scratch_types=[
            pltpu.VMEM((E_PER_W,), jnp.int32),
            pltpu.VMEM((N_NODES,), jnp.float32),
        ],
        compiler_params=params,
    )
    sc_tsum = pl.kernel(
        _sc_tsum_body,
        mesh=mesh,
        out_type=jax.ShapeDtypeStruct((NW, N_NODES), jnp.float32),
        scratch_types=[
            pltpu.VMEM((E_PER_W,), jnp.int32),
            pltpu.VMEM((E_PER_W,), jnp.int32),
            pltpu.VMEM((N_NODES,), jnp.float32),
            pltpu.VMEM((N_NODES,), jnp.float32),
        ],
        compiler_params=params,
    )
    return sc_degree, sc_tsum


def kernel(x, edge_index, W, b, Wd, bd):
    sc_degree, sc_tsum = _build_sc_kernels()
    src = edge_index[0].astype(jnp.int32)
    dst = edge_index[1].astype(jnp.int32)

    deg_part = sc_degree(dst)
    dinv = pl.pallas_call(
        _tc_dinv,
        out_shape=jax.ShapeDtypeStruct((1, N_NODES), jnp.float32),
    )(deg_part)

    t_part = sc_tsum(src, dst, dinv.reshape(N_NODES))

    out = pl.pallas_call(
        _tc_final,
        out_shape=jax.ShapeDtypeStruct((1, 10), jnp.float32),
    )(t_part, dinv, x, W, b.reshape(1, -1), Wd, bd.reshape(1, -1))
    return out.reshape(10)


# speedup vs baseline: 155.9739x; 1.2617x over previous
import functools

import jax
import jax.numpy as jnp
from jax import lax
from jax.experimental import pallas as pl
from jax.experimental.pallas import tpu as pltpu
from jax.experimental.pallas import tpu_sc as plsc

N_NODES = 10000
N_EDGES = 320000
L = 16
NC, NS = 2, 16
NW = NC * NS
E_PER_W = N_EDGES // NW

def _sc_degree_body(dst_hbm, out_hbm, dst_v, deg_v, sem):
    wid = lax.axis_index("s") * NC + lax.axis_index("c")
    base = wid * E_PER_W
    copy = pltpu.async_copy(dst_hbm.at[pl.ds(base, E_PER_W)], dst_v, sem)

    zeros = jnp.zeros((L,), jnp.float32)

    @plsc.parallel_loop(0, N_NODES // L, unroll=5)
    def _(i):
        deg_v[pl.ds(i * L, L)] = zeros

    copy.wait()
    ones = jnp.ones((L,), jnp.float32)

    @plsc.parallel_loop(0, E_PER_W // L, unroll=5)
    def _(i):
        idx = dst_v[pl.ds(i * L, L)]
        plsc.addupdate_scatter(deg_v, [idx], ones)

    pltpu.sync_copy(deg_v, out_hbm.at[wid])


def _sc_tsum_body(src_hbm, dst_hbm, dinv_hbm, out_hbm, src_v, dst_v, dinv_v, t_v, sem):
    wid = lax.axis_index("s") * NC + lax.axis_index("c")
    base = wid * E_PER_W
    c0 = pltpu.async_copy(dinv_hbm, dinv_v, sem)
    c1 = pltpu.async_copy(src_hbm.at[pl.ds(base, E_PER_W)], src_v, sem)
    c2 = pltpu.async_copy(dst_hbm.at[pl.ds(base, E_PER_W)], dst_v, sem)

    zeros = jnp.zeros((L,), jnp.float32)

    @plsc.parallel_loop(0, N_NODES // L, unroll=5)
    def _(i):
        t_v[pl.ds(i * L, L)] = zeros

    c0.wait()
    c1.wait()
    c2.wait()

    @plsc.parallel_loop(0, E_PER_W // L, unroll=5)
    def _(i):
        si = src_v[pl.ds(i * L, L)]
        di = dst_v[pl.ds(i * L, L)]
        dvals = plsc.load_gather(dinv_v, [di])
        plsc.addupdate_scatter(t_v, [si], dvals)

    pltpu.sync_copy(t_v, out_hbm.at[wid])


def _tc_dinv(deg_part_ref, out_ref):
    deg = jnp.sum(deg_part_ref[...], axis=0, keepdims=True) + 1.0
    out_ref[...] = lax.rsqrt(deg)


def _tc_final(t_part_ref, dinv_ref, x_ref, w_ref, b_ref, wd_ref, bd_ref, out_ref):
    t = jnp.sum(t_part_ref[...], axis=0, keepdims=True)
    dinv = dinv_ref[...]
    c = dinv * (dinv + t)
    cx = lax.dot_general(c, x_ref[...], (((1,), (0,)), ((), ())),
                         preferred_element_type=jnp.float32)
    pooled = lax.dot_general(cx, w_ref[...], (((1,), (0,)), ((), ())),
                             preferred_element_type=jnp.float32)
    pooled = pooled + float(N_NODES) * b_ref[...]
    logits = lax.dot_general(pooled, wd_ref[...], (((1,), (0,)), ((), ())),
                             preferred_element_type=jnp.float32)
    logits = logits + bd_ref[...]
    m = jnp.max(logits, axis=1, keepdims=True)
    e = jnp.exp(logits - m)
    out_ref[...] = e / jnp.sum(e, axis=1, keepdims=True)


@functools.cache
def _build_sc_kernels():
    mesh = plsc.VectorSubcoreMesh(core_axis_name="c", subcore_axis_name="s")
    params = pltpu.CompilerParams(needs_layout_passes=False)
    sc_degree = pl.kernel(
        _sc_degree_body,
        mesh=mesh,
        out_type=jax.ShapeDtypeStruct((NW, N_NODES), jnp.float32),
        scratch_types=[
            pltpu.VMEM((E_PER_W,), jnp.int32),
            pltpu.VMEM((N_NODES,), jnp.float32),
            pltpu.SemaphoreType.DMA,
        ],
        compiler_params=params,
    )
    sc_tsum = pl.kernel(
        _sc_tsum_body,
        mesh=mesh,
        out_type=jax.ShapeDtypeStruct((NW, N_NODES), jnp.float32),
        scratch_types=[
            pltpu.VMEM((E_PER_W,), jnp.int32),
            pltpu.VMEM((E_PER_W,), jnp.int32),
            pltpu.VMEM((N_NODES,), jnp.float32),
            pltpu.VMEM((N_NODES,), jnp.float32),
            pltpu.SemaphoreType.DMA,
        ],
        compiler_params=params,
    )
    return sc_degree, sc_tsum


def kernel(x, edge_index, W, b, Wd, bd):
    sc_degree, sc_tsum = _build_sc_kernels()
    src = edge_index[0].astype(jnp.int32)
    dst = edge_index[1].astype(jnp.int32)

    deg_part = sc_degree(dst)
    dinv = pl.pallas_call(
        _tc_dinv,
        out_shape=jax.ShapeDtypeStruct((1, N_NODES), jnp.float32),
    )(deg_part)

    t_part = sc_tsum(src, dst, dinv.reshape(N_NODES))

    out = pl.pallas_call(
        _tc_final,
        out_shape=jax.ShapeDtypeStruct((1, 10), jnp.float32),
    )(t_part, dinv, x, W, b.reshape(1, -1), Wd, bd.reshape(1, -1))
    return out.reshape(10)
